# baseline (speedup 1.0000x reference)
import functools

import jax
import jax.numpy as jnp
from jax import lax
from jax.experimental import pallas as pl
from jax.experimental.pallas import tpu as pltpu
from jax.experimental.pallas import tpu_sc as plsc

N_CLS = 1024
L_TXT = 61
N_CTX = 16
SEQ = 77
D_MODEL = 512
NW = 32
CPW = N_CLS // NW
LANES = 16


L_PAD = 64
SEQ_PAD = 80


def _sc_body(prompt_hbm, ptb_hbm, ctx_hbm, table_hbm, out_hbm, eos_hbm,
             idx_v, seq_v, pt_v, eos_v, sem):
    num_cores = 2
    wid = lax.axis_index("s") * num_cores + lax.axis_index("c")
    base = wid * CPW

    pltpu.sync_copy(prompt_hbm.at[pl.ds(base, CPW)], idx_v)
    pltpu.sync_copy(ptb_hbm.at[wid], pt_v)

    for g in range(CPW // LANES):
        def jbody(j, carry):
            m, am = carry
            v = pt_v[j, pl.ds(g * LANES, LANES)]
            upd = v > m
            return jnp.maximum(m, v), jnp.where(upd, j, am)

        m0 = jnp.full((LANES,), jnp.iinfo(jnp.int32).min, jnp.int32)
        am0 = jnp.zeros((LANES,), jnp.int32)
        _, am = lax.fori_loop(0, L_TXT, jbody, (m0, am0))
        eos_v[pl.ds(g * LANES, LANES)] = am + N_CTX
    pltpu.sync_copy(eos_v, eos_hbm.at[pl.ds(base, CPW)])

    def cbody(i, carry):
        c = base + i
        pltpu.async_copy(table_hbm.at[idx_v.at[i]],
                         seq_v.at[pl.ds(N_CTX, L_PAD)], sem).wait()
        for k in range(D_MODEL // LANES):
            seq_v[0, pl.ds(k * LANES, LANES)] = seq_v[N_CTX,
                                                      pl.ds(k * LANES, LANES)]
        pltpu.sync_copy(ctx_hbm.at[c], seq_v.at[pl.ds(1, N_CTX)])
        pltpu.sync_copy(seq_v.at[pl.ds(0, SEQ)], out_hbm.at[c])
        return carry

    lax.fori_loop(0, CPW, cbody, 0)


@functools.partial(
    pl.kernel,
    mesh=plsc.VectorSubcoreMesh(core_axis_name="c", subcore_axis_name="s"),
    compiler_params=pltpu.CompilerParams(use_tc_tiling_on_sc=False),
    out_type=(
        jax.ShapeDtypeStruct((N_CLS, SEQ, D_MODEL), jnp.float32),
        jax.ShapeDtypeStruct((N_CLS,), jnp.int32),
    ),
    scratch_types=[
        pltpu.VMEM((CPW, L_PAD), jnp.int32),
        pltpu.VMEM((SEQ_PAD, D_MODEL), jnp.float32),
        pltpu.VMEM((L_TXT, CPW), jnp.int32),
        pltpu.VMEM((CPW,), jnp.int32),
        pltpu.SemaphoreType.DMA,
    ],
)
def _prompt_learner_sc(prompt_hbm, ptb_hbm, ctx_hbm, table_hbm,
                       out_hbm, eos_hbm, idx_v, seq_v, pt_v, eos_v, sem):
    _sc_body(prompt_hbm, ptb_hbm, ctx_hbm, table_hbm, out_hbm, eos_hbm,
             idx_v, seq_v, pt_v, eos_v, sem)


def kernel(prompt, ctx_embedding, token_embedding):
    prompt_pad = jnp.pad(prompt, ((0, 0), (0, L_PAD - L_TXT)))
    ptb = jnp.transpose(prompt.reshape(NW, CPW, L_TXT), (0, 2, 1))
    token, eos = _prompt_learner_sc(prompt_pad, ptb, ctx_embedding,
                                    token_embedding)
    return (token, eos)
